# baseline (speedup 1.0000x reference)
import jax
import jax.numpy as jnp
from jax.experimental import pallas as pl

_K = 20
_VLAD_K = 50
_N_ITER = 10


def _assign_body(f_ref, cb_ref, f2_ref, c2_ref, lab_ref, cnt_ref, sid_ref,
                 perm_ref):
    f = f_ref[...]
    cb = cb_ref[...]
    n = f.shape[0]
    mm = jax.lax.dot_general(
        f, cb, (((1,), (1,)), ((), ())),
        preferred_element_type=jnp.float32)
    d = (f2_ref[...] - 2.0 * mm) + c2_ref[...]
    lab = jnp.argmin(d, axis=1, keepdims=True).astype(jnp.int32)
    lab_ref[...] = lab

    k_iota = jax.lax.broadcasted_iota(jnp.int32, (n, _K), 1)
    onehot = (lab == k_iota).astype(jnp.float32)
    counts = jnp.sum(onehot, axis=0, keepdims=True)
    cnt_ref[...] = counts

    r0 = jax.lax.broadcasted_iota(jnp.int32, (n, n), 0)
    r1 = jax.lax.broadcasted_iota(jnp.int32, (n, n), 1)
    lower = (r1 < r0).astype(jnp.float32)
    prev = jax.lax.dot_general(
        lower, onehot, (((1,), (0,)), ((), ())),
        preferred_element_type=jnp.float32)
    rank = jnp.sum(prev * onehot, axis=1, keepdims=True)

    c0 = jax.lax.broadcasted_iota(jnp.int32, (_K, _K), 0)
    c1 = jax.lax.broadcasted_iota(jnp.int32, (_K, _K), 1)
    tri = (c0 <= c1).astype(jnp.float32)
    cum = jax.lax.dot_general(
        counts, tri, (((1,), (0,)), ((), ())),
        preferred_element_type=jnp.float32)
    offs = cum - counts
    pos = rank + jnp.sum(onehot * offs, axis=1, keepdims=True)
    pos_i = pos.astype(jnp.int32)

    p_iota = jax.lax.broadcasted_iota(jnp.int32, (n, 1), 0).astype(jnp.float32)
    sid_ref[...] = jnp.sum((p_iota >= cum).astype(jnp.int32), axis=1,
                           keepdims=True)

    lane = jax.lax.broadcasted_iota(jnp.int32, (n, n), 1)
    m = (pos_i == lane).astype(jnp.float32)
    j_row = jax.lax.broadcasted_iota(jnp.int32, (1, n), 1).astype(jnp.float32)
    perm = jax.lax.dot_general(
        j_row, m, (((1,), (0,)), ((), ())),
        preferred_element_type=jnp.float32)
    perm_ref[...] = perm.astype(jnp.int32)


def _pallas_assign(feature, cents, f2):
    n, dd = feature.shape
    cb = cents.astype(jnp.bfloat16)
    c2 = jnp.sum(cents * cents, axis=1)
    lab, cnt, sid, perm = pl.pallas_call(
        _assign_body,
        out_shape=(jax.ShapeDtypeStruct((n, 1), jnp.int32),
                   jax.ShapeDtypeStruct((1, _K), jnp.float32),
                   jax.ShapeDtypeStruct((n, 1), jnp.int32),
                   jax.ShapeDtypeStruct((1, n), jnp.int32)),
    )(feature, cb, f2[:, None], c2[None, :])
    return lab[:, 0], cnt[0], sid[:, 0], perm[0]


def kernel(x):
    b, c, h, w = x.shape
    n = c
    y = x.reshape(b, c, h * w)
    f2 = jnp.sum(y * y, axis=2)
    y_flat = y.reshape(b * n, h * w)
    base_k = (_K * jnp.arange(b, dtype=jnp.int32))[:, None]
    base_n = (n * jnp.arange(b, dtype=jnp.int32))[:, None]
    assign = jax.vmap(_pallas_assign, in_axes=(0, 0, 0))

    def step(cents, _):
        _, counts, sid, perm = assign(y, cents, f2)
        keys = (sid + base_k).reshape(b * n)
        gperm = (perm + base_n).reshape(b * n)
        y_sorted = jnp.take(y_flat, gperm, axis=0)
        sums = (jnp.zeros((b * _K, h * w), y.dtype)
                .at[keys].add(y_sorted, indices_are_sorted=True)
                .reshape(b, _K, h * w))
        new = jnp.where(counts[:, :, None] > 0,
                        sums / jnp.maximum(counts, 1.0)[:, :, None],
                        cents)
        return new, None

    cents0 = y[:, :_K, :]
    cents, _ = jax.lax.scan(step, cents0, None, length=_N_ITER)

    labels, _, _, _ = assign(y, cents, f2)

    def finish(y_i, cents_i, lab_i):
        resid = y_i - cents_i[lab_i]
        return jnp.zeros((_VLAD_K, y_i.shape[1]), y_i.dtype).at[lab_i].add(resid)

    return jax.vmap(finish)(y, cents, labels)


# speedup vs baseline: 1.1861x; 1.1861x over previous
import jax
import jax.numpy as jnp
from jax.experimental import pallas as pl

_K = 20
_VLAD_K = 50
_N_ITER = 10


def _labels_body(f_ref, cb_ref, f2_ref, c2_ref, lab_ref, cnt_ref):
    f = f_ref[...]
    cb = cb_ref[...]
    n = f.shape[0]
    mm = jax.lax.dot_general(
        f, cb, (((1,), (1,)), ((), ())),
        preferred_element_type=jnp.float32)
    d = (f2_ref[...] - 2.0 * mm) + c2_ref[...]
    lab = jnp.argmin(d, axis=1, keepdims=True).astype(jnp.int32)
    lab_ref[...] = lab
    k_iota = jax.lax.broadcasted_iota(jnp.int32, (n, _K), 1)
    onehot = (lab == k_iota).astype(jnp.float32)
    cnt_ref[...] = jnp.sum(onehot, axis=0, keepdims=True)


def _pallas_assign(feature, cents, f2):
    n, dd = feature.shape
    cb = cents.astype(jnp.bfloat16)
    c2 = jnp.sum(cents * cents, axis=1)
    labels, counts = pl.pallas_call(
        _labels_body,
        out_shape=(jax.ShapeDtypeStruct((n, 1), jnp.int32),
                   jax.ShapeDtypeStruct((1, _K), jnp.float32)),
    )(feature, cb, f2[:, None], c2[None, :])
    return labels[:, 0], counts[0]


def _vlad_one(y_i):
    n = y_i.shape[0]
    f2 = jnp.sum(y_i * y_i, axis=1)
    cents0 = y_i[:_K]

    def step(cents, _):
        labels, counts = _pallas_assign(y_i, cents, f2)
        sums = jax.ops.segment_sum(y_i, labels, num_segments=_K)
        new = jnp.where(counts[:, None] > 0,
                        sums / jnp.maximum(counts, 1.0)[:, None],
                        cents)
        return new, None

    cents, _ = jax.lax.scan(step, cents0, None, length=_N_ITER)
    labels, _ = _pallas_assign(y_i, cents, f2)
    resid = y_i - cents[labels]
    out = jnp.zeros((_VLAD_K, y_i.shape[1]), y_i.dtype).at[labels].add(resid)
    return out


def kernel(x):
    b, c, h, w = x.shape
    y = x.reshape(b, c, h * w)
    return jax.vmap(_vlad_one)(y)
